# Initial kernel scaffold; baseline (speedup 1.0000x reference)
import jax
import jax.numpy as jnp
from jax.experimental import pallas as pl


def kernel(x, weight):
    raise NotImplementedError("write your pallas kernel here")


# speedup vs baseline: 2.9902x; 2.9902x over previous
import functools

import jax
import jax.numpy as jnp
from jax import lax
from jax.experimental import pallas as pl
from jax.experimental.pallas import tpu as pltpu
from jax.experimental.pallas import tpu_sc as plsc

BATCH = 16384
FIELDS = 26
DIM = 128
TOTAL_ROWS = BATCH * FIELDS
NUM_CORES = 2
NUM_SUBCORES = 16
NUM_WORKERS = NUM_CORES * NUM_SUBCORES
ROWS_PER_WORKER = TOTAL_ROWS // NUM_WORKERS
GATHER_ROWS = 128
GATHERS_PER_WORKER = ROWS_PER_WORKER // GATHER_ROWS
IDX_BLOCKS = TOTAL_ROWS // GATHER_ROWS

_mesh = plsc.VectorSubcoreMesh(core_axis_name="c", subcore_axis_name="s")


@functools.partial(
    pl.kernel,
    mesh=_mesh,
    out_type=jax.ShapeDtypeStruct((TOTAL_ROWS, DIM), jnp.float32),
    scratch_types=[
        pltpu.VMEM((GATHERS_PER_WORKER, GATHER_ROWS), jnp.int32),
        pltpu.VMEM((GATHER_ROWS, DIM), jnp.float32),
        pltpu.SemaphoreType.DMA,
    ],
)
def _sc_gather(idx_hbm, table_hbm, out_hbm, idx_v, rows_v, sem):
    wid = lax.axis_index("s") * NUM_CORES + lax.axis_index("c")
    blk_base = wid * GATHERS_PER_WORKER
    pltpu.sync_copy(idx_hbm.at[pl.ds(blk_base, GATHERS_PER_WORKER)], idx_v)

    def body(g, carry):
        pltpu.async_copy(table_hbm.at[idx_v.at[g]], rows_v, sem).wait()
        pltpu.sync_copy(
            rows_v, out_hbm.at[pl.ds((blk_base + g) * GATHER_ROWS, GATHER_ROWS)]
        )
        return carry

    lax.fori_loop(0, GATHERS_PER_WORKER, body, 0)


def kernel(x, weight):
    idx = x.reshape(IDX_BLOCKS, GATHER_ROWS)
    out = _sc_gather(idx, weight)
    return out.reshape(BATCH, FIELDS, DIM)


# speedup vs baseline: 3.3906x; 1.1339x over previous
import functools

import jax
import jax.numpy as jnp
from jax import lax
from jax.experimental import pallas as pl
from jax.experimental.pallas import tpu as pltpu
from jax.experimental.pallas import tpu_sc as plsc

BATCH = 16384
FIELDS = 26
DIM = 128
TOTAL_ROWS = BATCH * FIELDS
NUM_CORES = 2
NUM_SUBCORES = 16
NUM_WORKERS = NUM_CORES * NUM_SUBCORES
ROWS_PER_WORKER = TOTAL_ROWS // NUM_WORKERS
GATHER_ROWS = 128
BLOCKS_PER_WORKER = ROWS_PER_WORKER // GATHER_ROWS
IDX_BLOCKS = TOTAL_ROWS // GATHER_ROWS
NB = 2
NGROUPS = BLOCKS_PER_WORKER // NB

_mesh = plsc.VectorSubcoreMesh(core_axis_name="c", subcore_axis_name="s")


@functools.partial(
    pl.kernel,
    mesh=_mesh,
    out_type=jax.ShapeDtypeStruct((TOTAL_ROWS, DIM), jnp.float32),
    scratch_types=[
        pltpu.VMEM((BLOCKS_PER_WORKER, GATHER_ROWS), jnp.int32),
        pltpu.VMEM((NB, GATHER_ROWS, DIM), jnp.float32),
        pltpu.VMEM((NB, GATHER_ROWS, DIM), jnp.float32),
        pltpu.SemaphoreType.DMA,
        pltpu.SemaphoreType.DMA,
        pltpu.SemaphoreType.DMA,
        pltpu.SemaphoreType.DMA,
    ],
)
def _sc_gather(idx_hbm, table_hbm, out_hbm, idx_v, rows0, rows1,
               gsem0, gsem1, wsem0, wsem1):
    wid = lax.axis_index("s") * NUM_CORES + lax.axis_index("c")
    blk_base = wid * BLOCKS_PER_WORKER
    rows = (rows0, rows1)
    gsem = (gsem0, gsem1)
    wsem = (wsem0, wsem1)

    pltpu.sync_copy(idx_hbm.at[pl.ds(blk_base, BLOCKS_PER_WORKER)], idx_v)

    def start_gathers(group, s):
        for b in range(NB):
            blk = group * NB + b
            pltpu.async_copy(table_hbm.at[idx_v.at[blk]], rows[s].at[b], gsem[s])

    def wait_gathers(s):
        for b in range(NB):
            pltpu.make_async_copy(
                out_hbm.at[pl.ds(0, GATHER_ROWS)], rows[s].at[b], gsem[s]
            ).wait()

    def start_writes(group, s):
        for b in range(NB):
            blk = group * NB + b
            pltpu.async_copy(
                rows[s].at[b],
                out_hbm.at[pl.ds((blk_base + blk) * GATHER_ROWS, GATHER_ROWS)],
                wsem[s],
            )

    def wait_writes(s):
        for b in range(NB):
            pltpu.make_async_copy(
                rows[s].at[b], out_hbm.at[pl.ds(0, GATHER_ROWS)], wsem[s]
            ).wait()

    start_gathers(0, 0)

    def body(i, carry):
        h0 = 2 * i
        @pl.when(h0 >= 1)
        def _():
            wait_writes(1)

        start_gathers(h0 + 1, 1)
        wait_gathers(0)
        start_writes(h0, 0)
        wait_writes(0)

        @pl.when(h0 + 2 < NGROUPS)
        def _():
            start_gathers(h0 + 2, 0)

        wait_gathers(1)
        start_writes(h0 + 1, 1)
        return carry

    lax.fori_loop(0, NGROUPS // 2, body, 0)
    wait_writes(1)


def kernel(x, weight):
    idx = x.reshape(IDX_BLOCKS, GATHER_ROWS)
    out = _sc_gather(idx, weight)
    return out.reshape(BATCH, FIELDS, DIM)
